# baseline (speedup 1.0000x reference)
import functools

import jax
import jax.numpy as jnp
from jax import lax
from jax.experimental import pallas as pl
from jax.experimental.pallas import tpu as pltpu
from jax.experimental.pallas import tpu_sc as plsc

_NC = 2
_NS = 16
_NW = _NC * _NS
_CH = 80


def _lrelu(v):
    return jnp.where(v > 0, v, 0.01 * v)


def _node_proj_body(x_ref, wr_ref, br_ref, w1a_ref, b1_ref, o_ref):
    xr = jnp.dot(x_ref[...], wr_ref[...], preferred_element_type=jnp.float32)
    xr = xr + br_ref[...]
    o_ref[...] = (
        jnp.dot(xr, w1a_ref[...], preferred_element_type=jnp.float32) + b1_ref[...]
    )


def _node_proj(x, Wr, br, W1a, b1, block_n=2000):
    n, d = x.shape
    h = W1a.shape[1]
    grid = n // block_n
    full = lambda i: (0, 0)
    return pl.pallas_call(
        _node_proj_body,
        grid=(grid,),
        in_specs=[
            pl.BlockSpec((block_n, d), lambda i: (i, 0)),
            pl.BlockSpec((d, d), full),
            pl.BlockSpec((1, d), full),
            pl.BlockSpec((d, h), full),
            pl.BlockSpec((1, h), full),
        ],
        out_specs=pl.BlockSpec((block_n, h), lambda i: (i, 0)),
        out_shape=jax.ShapeDtypeStruct((n, h), jnp.float32),
    )(x, Wr, br.reshape(1, d), W1a, b1.reshape(1, h))


def _edge_mlp_body(pg_ref, hm_ref, dy_ref, w1b_ref, w2_ref, b2_ref, w3_ref,
                   b3_ref, w4_ref, b4_ref, w5_ref, b5_ref, o_ref):
    f32 = jnp.float32
    bf16 = jnp.bfloat16
    hid = w1b_ref.shape[1]
    h1 = _lrelu(
        pg_ref[...][:, :hid]
        + jnp.dot(hm_ref[...].astype(bf16), w1b_ref[...].astype(bf16),
                  preferred_element_type=f32)
    )
    h2 = _lrelu(jnp.dot(h1.astype(bf16), w2_ref[...].astype(bf16),
                        preferred_element_type=f32) + b2_ref[...])
    h3 = _lrelu(jnp.dot(h2.astype(bf16), w3_ref[...].astype(bf16),
                        preferred_element_type=f32) + b3_ref[...])
    h4 = _lrelu(jnp.dot(h3.astype(bf16), w4_ref[...].astype(bf16),
                        preferred_element_type=f32) + b4_ref[...])
    m = jnp.dot(h4.astype(bf16), w5_ref[...].astype(bf16),
                preferred_element_type=f32) + b5_ref[...]
    o_ref[...] = m * dy_ref[...]


def _edge_mlp(pg, hm, dy, W1b, W2, b2, W3, b3, W4, b4, W5, b5, block_e=2000):
    e = hm.shape[0]
    dh = hm.shape[1]
    hid = W2.shape[0]
    out_d = W5.shape[1]
    grid = e // block_e
    full = lambda i: (0, 0)
    return pl.pallas_call(
        _edge_mlp_body,
        grid=(grid,),
        in_specs=[
            pl.BlockSpec((block_e, pg.shape[1]), lambda i: (i, 0)),
            pl.BlockSpec((block_e, dh), lambda i: (i, 0)),
            pl.BlockSpec((block_e, 1), lambda i: (i, 0)),
            pl.BlockSpec((dh, hid), full),
            pl.BlockSpec((hid, hid), full),
            pl.BlockSpec((1, hid), full),
            pl.BlockSpec((hid, hid), full),
            pl.BlockSpec((1, hid), full),
            pl.BlockSpec((hid, hid), full),
            pl.BlockSpec((1, hid), full),
            pl.BlockSpec((hid, out_d), full),
            pl.BlockSpec((1, out_d), full),
        ],
        out_specs=pl.BlockSpec((block_e, out_d), lambda i: (i, 0)),
        out_shape=jax.ShapeDtypeStruct((e, out_d), jnp.float32),
    )(pg, hm, dy, W1b, W2, b2.reshape(1, hid), W3, b3.reshape(1, hid), W4,
      b4.reshape(1, hid), W5, b5.reshape(1, out_d))


def _sc_gather(table, idx):
    n, d = table.shape
    e = idx.shape[0]
    epw = e // _NW
    steps = epw // _CH
    mesh = plsc.VectorSubcoreMesh(
        core_axis_name="c", subcore_axis_name="s", num_cores=_NC,
        num_subcores=_NS)

    @functools.partial(
        pl.kernel,
        out_type=jax.ShapeDtypeStruct((e, d), jnp.float32),
        mesh=mesh,
        scratch_types=[
            pltpu.VMEM((_CH,), jnp.int32),
            pltpu.VMEM((_CH, d), jnp.float32),
            pltpu.SemaphoreType.DMA,
        ],
    )
    def gk(table_hbm, idx_hbm, out_hbm, idx_v, rows_v, sem):
        wid = lax.axis_index("s") * _NC + lax.axis_index("c")
        base = wid * epw

        def body(i, carry):
            off = pl.multiple_of(base + i * _CH, _CH)
            pltpu.sync_copy(idx_hbm.at[pl.ds(off, _CH)], idx_v)
            pltpu.async_copy(table_hbm.at[idx_v], rows_v, sem).wait()
            pltpu.sync_copy(rows_v, out_hbm.at[pl.ds(off, _CH)])
            return carry

        lax.fori_loop(0, steps, body, 0)

    return gk(table, idx)


def _sc_scatter(m, idx, n):
    e, d = m.shape
    epw = e // _NW
    steps = epw // _CH
    npt = 640
    n_pad = _NS * npt
    mesh = plsc.VectorSubcoreMesh(
        core_axis_name="c", subcore_axis_name="s", num_cores=_NC,
        num_subcores=_NS)

    @functools.partial(
        pl.kernel,
        out_type=(
            jax.ShapeDtypeStruct((_NC, n_pad, d), jnp.float32),
            jax.ShapeDtypeStruct((_NC * n_pad,), jnp.float32),
        ),
        mesh=mesh,
        scratch_types=[
            pltpu.VMEM((_CH,), jnp.int32),
            pltpu.VMEM((_CH, d), jnp.float32),
            pltpu.VMEM((_CH,), jnp.float32),
            pltpu.VMEM((_CH,), jnp.float32),
            pltpu.VMEM_SHARED((n_pad, d), jnp.float32),
            pltpu.VMEM_SHARED((n_pad,), jnp.float32),
        ],
    )
    def sk(m_hbm, idx_hbm, sum_hbm, cnt_hbm, idx_v, rows_v, ones_v, z1_v,
           table_sh, cnt_sh):
        cid = lax.axis_index("c")
        sid = lax.axis_index("s")
        wid = sid * _NC + cid
        base = wid * epw

        zro = jnp.zeros((16,), jnp.float32)
        one = jnp.ones((16,), jnp.float32)

        def fill_body(r, carry):
            for j in range(d // 16):
                rows_v[r, pl.ds(j * 16, 16)] = zro
            return carry

        lax.fori_loop(0, _CH, fill_body, 0)
        for j in range(_CH // 16):
            ones_v[pl.ds(j * 16, 16)] = one
            z1_v[pl.ds(j * 16, 16)] = zro

        zbase = sid * npt
        for j in range(npt // _CH):
            pltpu.sync_copy(rows_v, table_sh.at[pl.ds(zbase + j * _CH, _CH)])
            pltpu.sync_copy(z1_v, cnt_sh.at[pl.ds(zbase + j * _CH, _CH)])
        plsc.subcore_barrier()

        def body(i, carry):
            off = pl.multiple_of(base + i * _CH, 16)
            pltpu.sync_copy(idx_hbm.at[pl.ds(off, _CH)], idx_v)
            pltpu.sync_copy(m_hbm.at[pl.ds(off, _CH)], rows_v)
            pltpu.sync_copy(rows_v, table_sh.at[idx_v], add=True)
            pltpu.sync_copy(ones_v, cnt_sh.at[idx_v], add=True)
            return carry

        lax.fori_loop(0, steps, body, 0)
        plsc.subcore_barrier()

        for j in range(npt // _CH):
            r0 = zbase + j * _CH
            pltpu.sync_copy(table_sh.at[pl.ds(r0, _CH)], rows_v)
            pltpu.sync_copy(rows_v, sum_hbm.at[cid, pl.ds(r0, _CH)])
            pltpu.sync_copy(cnt_sh.at[pl.ds(r0, _CH)], z1_v)
            pltpu.sync_copy(z1_v, cnt_hbm.at[pl.ds(cid * n_pad + r0, _CH)])

    return sk(m, idx)


def _combine_body(s_ref, c0_ref, c1_ref, o_ref):
    s = s_ref[0] + s_ref[1]
    c = c0_ref[...] + c1_ref[...]
    o_ref[...] = s / jnp.maximum(c, 1.0)


def _combine(sums, cnts, n, block_n=2000):
    d = sums.shape[2]
    n_pad = sums.shape[1]
    c0 = cnts[:n_pad].reshape(n_pad, 1)
    c1 = cnts[n_pad:].reshape(n_pad, 1)
    grid = n // block_n
    return pl.pallas_call(
        _combine_body,
        grid=(grid,),
        in_specs=[
            pl.BlockSpec((2, block_n, d), lambda i: (0, i, 0)),
            pl.BlockSpec((block_n, 1), lambda i: (i, 0)),
            pl.BlockSpec((block_n, 1), lambda i: (i, 0)),
        ],
        out_specs=pl.BlockSpec((block_n, d), lambda i: (i, 0)),
        out_shape=jax.ShapeDtypeStruct((n, d), jnp.float32),
    )(sums, c0, c1)


def kernel(x, edge_index, h_mat_edge, delta_y, Wr, br, W1, b1, W2, b2, W3, b3,
           W4, b4, W5, b5):
    n = x.shape[0]
    d = x.shape[1]
    W1a = W1[:d]
    W1b = W1[d:]

    hid = W1a.shape[1]
    W1a_p = jnp.concatenate([W1a, jnp.zeros((d, d - hid), jnp.float32)], axis=1)
    b1_p = jnp.concatenate([b1, jnp.zeros((d - hid,), jnp.float32)])
    p = _node_proj(x, Wr, br, W1a_p, b1_p)

    dst = edge_index[1]
    pg = _sc_gather(p, dst)

    m = _edge_mlp(pg, h_mat_edge, delta_y, W1b, W2, b2, W3, b3, W4, b4, W5, b5)

    sums, cnts = _sc_scatter(m, dst, n)
    return _combine(sums, cnts, n)


# speedup vs baseline: 1.3751x; 1.3751x over previous
import functools

import jax
import jax.numpy as jnp
from jax import lax
from jax.experimental import pallas as pl
from jax.experimental.pallas import tpu as pltpu
from jax.experimental.pallas import tpu_sc as plsc

_NC = 2
_NS = 16
_NW = _NC * _NS
_CH = 80


def _lrelu(v):
    return jnp.where(v > 0, v, 0.01 * v)


def _node_proj_body(x_ref, wr_ref, br_ref, w1a_ref, b1_ref, o_ref):
    xr = jnp.dot(x_ref[...], wr_ref[...], preferred_element_type=jnp.float32)
    xr = xr + br_ref[...]
    o_ref[...] = (
        jnp.dot(xr, w1a_ref[...], preferred_element_type=jnp.float32) + b1_ref[...]
    )


def _node_proj(x, Wr, br, W1a, b1, block_n=2000):
    n, d = x.shape
    h = W1a.shape[1]
    grid = n // block_n
    full = lambda i: (0, 0)
    return pl.pallas_call(
        _node_proj_body,
        grid=(grid,),
        in_specs=[
            pl.BlockSpec((block_n, d), lambda i: (i, 0)),
            pl.BlockSpec((d, d), full),
            pl.BlockSpec((1, d), full),
            pl.BlockSpec((d, h), full),
            pl.BlockSpec((1, h), full),
        ],
        out_specs=pl.BlockSpec((block_n, h), lambda i: (i, 0)),
        out_shape=jax.ShapeDtypeStruct((n, h), jnp.float32),
    )(x, Wr, br.reshape(1, d), W1a, b1.reshape(1, h))


def _edge_mlp_body(pg_ref, hm_ref, dy_ref, w1b_ref, w2_ref, b2_ref, w3_ref,
                   b3_ref, w4_ref, b4_ref, w5_ref, b5_ref, o_ref):
    f32 = jnp.float32
    hid = w1b_ref.shape[1]
    h1 = _lrelu(
        pg_ref[...][:, :hid]
        + jnp.dot(hm_ref[...], w1b_ref[...], preferred_element_type=f32)
    )
    h2 = _lrelu(jnp.dot(h1, w2_ref[...], preferred_element_type=f32) + b2_ref[...])
    h3 = _lrelu(jnp.dot(h2, w3_ref[...], preferred_element_type=f32) + b3_ref[...])
    h4 = _lrelu(jnp.dot(h3, w4_ref[...], preferred_element_type=f32) + b4_ref[...])
    m = jnp.dot(h4, w5_ref[...], preferred_element_type=f32) + b5_ref[...]
    o_ref[...] = m * dy_ref[...]


def _edge_mlp(pg, hm, dy, W1b, W2, b2, W3, b3, W4, b4, W5, b5, block_e=2000):
    e = hm.shape[0]
    dh = hm.shape[1]
    hid = W2.shape[0]
    out_d = W5.shape[1]
    grid = e // block_e
    full = lambda i: (0, 0)
    return pl.pallas_call(
        _edge_mlp_body,
        grid=(grid,),
        in_specs=[
            pl.BlockSpec((block_e, pg.shape[1]), lambda i: (i, 0)),
            pl.BlockSpec((block_e, dh), lambda i: (i, 0)),
            pl.BlockSpec((block_e, 1), lambda i: (i, 0)),
            pl.BlockSpec((dh, hid), full),
            pl.BlockSpec((hid, hid), full),
            pl.BlockSpec((1, hid), full),
            pl.BlockSpec((hid, hid), full),
            pl.BlockSpec((1, hid), full),
            pl.BlockSpec((hid, hid), full),
            pl.BlockSpec((1, hid), full),
            pl.BlockSpec((hid, out_d), full),
            pl.BlockSpec((1, out_d), full),
        ],
        out_specs=pl.BlockSpec((block_e, out_d), lambda i: (i, 0)),
        out_shape=jax.ShapeDtypeStruct((e, out_d), jnp.float32),
    )(pg, hm, dy, W1b, W2, b2.reshape(1, hid), W3, b3.reshape(1, hid), W4,
      b4.reshape(1, hid), W5, b5.reshape(1, out_d))


def _sc_gather(table, idx):
    n, d = table.shape
    e = idx.shape[0]
    epw = e // _NW
    steps = epw // _CH
    pairs = (steps - 1) // 2
    mesh = plsc.VectorSubcoreMesh(
        core_axis_name="c", subcore_axis_name="s", num_cores=_NC,
        num_subcores=_NS)

    @functools.partial(
        pl.kernel,
        out_type=jax.ShapeDtypeStruct((e, d), jnp.float32),
        mesh=mesh,
        scratch_types=[
            pltpu.VMEM((epw,), jnp.int32),
            pltpu.VMEM((_CH, d), jnp.float32),
            pltpu.VMEM((_CH, d), jnp.float32),
            pltpu.SemaphoreType.DMA,
            pltpu.SemaphoreType.DMA,
            pltpu.SemaphoreType.DMA,
        ],
    )
    def gk(table_hbm, idx_hbm, out_hbm, idx_all, rows0, rows1, semg, sems0,
           sems1):
        wid = lax.axis_index("s") * _NC + lax.axis_index("c")
        base = wid * epw
        pltpu.sync_copy(idx_hbm.at[pl.ds(pl.multiple_of(base, _CH), epw)],
                        idx_all)

        def gather(c, buf):
            pltpu.async_copy(table_hbm.at[idx_all.at[pl.ds(c * _CH, _CH)]],
                             buf, semg).wait()

        def store_start(c, buf, sem):
            off = pl.multiple_of(base + c * _CH, _CH)
            return pltpu.async_copy(buf, out_hbm.at[pl.ds(off, _CH)], sem)

        gather(0, rows0)

        def body(i, carry):
            c0 = 2 * i
            s0 = store_start(c0, rows0, sems0)
            gather(c0 + 1, rows1)
            s1 = store_start(c0 + 1, rows1, sems1)
            s0.wait()
            gather(c0 + 2, rows0)
            s1.wait()
            return carry

        lax.fori_loop(0, pairs, body, 0)
        store_start(steps - 1, rows0, sems0).wait()

    return gk(table, idx)


def _sc_scatter(m, idx, n):
    e, d = m.shape
    epw = e // _NW
    steps = epw // _CH
    npt = 640
    n_pad = _NS * npt
    mesh = plsc.VectorSubcoreMesh(
        core_axis_name="c", subcore_axis_name="s", num_cores=_NC,
        num_subcores=_NS)

    pairs = (steps - 1) // 2
    idx3d = idx.reshape(_NW, steps, _CH)

    @functools.partial(
        pl.kernel,
        out_type=(
            jax.ShapeDtypeStruct((_NC, n_pad, d), jnp.float32),
            jax.ShapeDtypeStruct((_NC * n_pad,), jnp.float32),
        ),
        mesh=mesh,
        scratch_types=[
            pltpu.VMEM((steps, _CH), jnp.int32),
            pltpu.VMEM((_CH, d), jnp.float32),
            pltpu.VMEM((_CH, d), jnp.float32),
            pltpu.VMEM((_CH,), jnp.float32),
            pltpu.VMEM((_CH,), jnp.float32),
            pltpu.VMEM_SHARED((n_pad, d), jnp.float32),
            pltpu.VMEM_SHARED((n_pad,), jnp.float32),
            pltpu.SemaphoreType.DMA,
            pltpu.SemaphoreType.DMA,
            pltpu.SemaphoreType.DMA,
            pltpu.SemaphoreType.DMA,
        ],
    )
    def sk(m_hbm, idx_hbm, sum_hbm, cnt_hbm, idx_all, rows0, rows1, ones_v,
           z1_v, table_sh, cnt_sh, semm, sema0, sema1, semo):
        cid = lax.axis_index("c")
        sid = lax.axis_index("s")
        wid = sid * _NC + cid
        base = wid * epw

        zro = jnp.zeros((16,), jnp.float32)
        one = jnp.ones((16,), jnp.float32)

        def fill_body(r, carry):
            for j in range(d // 16):
                rows0[r, pl.ds(j * 16, 16)] = zro
            return carry

        lax.fori_loop(0, _CH, fill_body, 0)
        for j in range(_CH // 16):
            ones_v[pl.ds(j * 16, 16)] = one
            z1_v[pl.ds(j * 16, 16)] = zro

        pltpu.sync_copy(idx_hbm.at[wid], idx_all)

        zbase = sid * npt
        for j in range(npt // _CH):
            pltpu.sync_copy(rows0, table_sh.at[pl.ds(zbase + j * _CH, _CH)])
            pltpu.sync_copy(z1_v, cnt_sh.at[pl.ds(zbase + j * _CH, _CH)])
        plsc.subcore_barrier()

        def load_m(c, buf):
            off = pl.multiple_of(base + c * _CH, 16)
            pltpu.sync_copy(m_hbm.at[pl.ds(off, _CH)], buf)

        def scat_start(c, buf, sem):
            d0 = pltpu.async_copy(buf, table_sh.at[idx_all.at[c]], sem,
                                  add=True)
            d1 = pltpu.async_copy(ones_v, cnt_sh.at[idx_all.at[c]], semo,
                                  add=True)
            return d0, d1

        load_m(0, rows0)

        def body(i, carry):
            c0 = 2 * i
            s0 = scat_start(c0, rows0, sema0)
            load_m(c0 + 1, rows1)
            s1 = scat_start(c0 + 1, rows1, sema1)
            for dsc in s0:
                dsc.wait()
            load_m(c0 + 2, rows0)
            for dsc in s1:
                dsc.wait()
            return carry

        lax.fori_loop(0, pairs, body, 0)
        for dsc in scat_start(steps - 1, rows0, sema0):
            dsc.wait()
        plsc.subcore_barrier()

        for j in range(npt // _CH):
            r0 = zbase + j * _CH
            pltpu.sync_copy(table_sh.at[pl.ds(r0, _CH)], rows0)
            pltpu.sync_copy(rows0, sum_hbm.at[cid, pl.ds(r0, _CH)])
            pltpu.sync_copy(cnt_sh.at[pl.ds(r0, _CH)], z1_v)
            pltpu.sync_copy(z1_v, cnt_hbm.at[pl.ds(cid * n_pad + r0, _CH)])

    return sk(m, idx3d)


def _combine_body(s_ref, c0_ref, c1_ref, o_ref):
    s = s_ref[0] + s_ref[1]
    c = c0_ref[...] + c1_ref[...]
    o_ref[...] = s / jnp.maximum(c, 1.0)


def _combine(sums, cnts, n, block_n=2000):
    d = sums.shape[2]
    n_pad = sums.shape[1]
    c0 = cnts[:n_pad].reshape(n_pad, 1)
    c1 = cnts[n_pad:].reshape(n_pad, 1)
    grid = n // block_n
    return pl.pallas_call(
        _combine_body,
        grid=(grid,),
        in_specs=[
            pl.BlockSpec((2, block_n, d), lambda i: (0, i, 0)),
            pl.BlockSpec((block_n, 1), lambda i: (i, 0)),
            pl.BlockSpec((block_n, 1), lambda i: (i, 0)),
        ],
        out_specs=pl.BlockSpec((block_n, d), lambda i: (i, 0)),
        out_shape=jax.ShapeDtypeStruct((n, d), jnp.float32),
    )(sums, c0, c1)


def kernel(x, edge_index, h_mat_edge, delta_y, Wr, br, W1, b1, W2, b2, W3, b3,
           W4, b4, W5, b5):
    n = x.shape[0]
    d = x.shape[1]
    W1a = W1[:d]
    W1b = W1[d:]

    hid = W1a.shape[1]
    W1a_p = jnp.concatenate([W1a, jnp.zeros((d, d - hid), jnp.float32)], axis=1)
    b1_p = jnp.concatenate([b1, jnp.zeros((d - hid,), jnp.float32)])
    p = _node_proj(x, Wr, br, W1a_p, b1_p)

    dst = edge_index[1]
    pg = _sc_gather(p, dst)

    m = _edge_mlp(pg, h_mat_edge, delta_y, W1b, W2, b2, W3, b3, W4, b4, W5, b5)

    sums, cnts = _sc_scatter(m, dst, n)
    return _combine(sums, cnts, n)


# speedup vs baseline: 1.4960x; 1.0880x over previous
import functools

import jax
import jax.numpy as jnp
from jax import lax
from jax.experimental import pallas as pl
from jax.experimental.pallas import tpu as pltpu
from jax.experimental.pallas import tpu_sc as plsc

_NC = 2
_NS = 16
_NW = _NC * _NS
_CH = 80


def _lrelu(v):
    return jnp.where(v > 0, v, 0.01 * v)


def _node_proj_body(x_ref, wr_ref, br_ref, w1a_ref, b1_ref, o_ref):
    xr = jnp.dot(x_ref[...], wr_ref[...], preferred_element_type=jnp.float32)
    xr = xr + br_ref[...]
    o_ref[...] = (
        jnp.dot(xr, w1a_ref[...], preferred_element_type=jnp.float32) + b1_ref[...]
    )


def _node_proj(x, Wr, br, W1a, b1, block_n=2000):
    n, d = x.shape
    h = W1a.shape[1]
    grid = n // block_n
    full = lambda i: (0, 0)
    return pl.pallas_call(
        _node_proj_body,
        grid=(grid,),
        in_specs=[
            pl.BlockSpec((block_n, d), lambda i: (i, 0)),
            pl.BlockSpec((d, d), full),
            pl.BlockSpec((1, d), full),
            pl.BlockSpec((d, h), full),
            pl.BlockSpec((1, h), full),
        ],
        out_specs=pl.BlockSpec((block_n, h), lambda i: (i, 0)),
        out_shape=jax.ShapeDtypeStruct((n, h), jnp.float32),
    )(x, Wr, br.reshape(1, d), W1a, b1.reshape(1, h))


def _edge_mlp_body(pg_ref, hm_ref, dy_ref, w1b_ref, w2_ref, b2_ref, w3_ref,
                   b3_ref, w4_ref, b4_ref, w5_ref, b5_ref, o_ref):
    f32 = jnp.float32
    hid = w1b_ref.shape[1]
    h1 = _lrelu(
        pg_ref[...][:, :hid]
        + jnp.dot(hm_ref[...], w1b_ref[...], preferred_element_type=f32)
    )
    h2 = _lrelu(jnp.dot(h1, w2_ref[...], preferred_element_type=f32) + b2_ref[...])
    h3 = _lrelu(jnp.dot(h2, w3_ref[...], preferred_element_type=f32) + b3_ref[...])
    h4 = _lrelu(jnp.dot(h3, w4_ref[...], preferred_element_type=f32) + b4_ref[...])
    m = jnp.dot(h4, w5_ref[...], preferred_element_type=f32) + b5_ref[...]
    o_ref[...] = m * dy_ref[...]


def _edge_mlp(pg, hm, dy, W1b, W2, b2, W3, b3, W4, b4, W5, b5, block_e=4000):
    e = hm.shape[0]
    dh = hm.shape[1]
    hid = W2.shape[0]
    out_d = W5.shape[1]
    grid = e // block_e
    full = lambda i: (0, 0)
    return pl.pallas_call(
        _edge_mlp_body,
        grid=(grid,),
        in_specs=[
            pl.BlockSpec((block_e, pg.shape[1]), lambda i: (i, 0)),
            pl.BlockSpec((block_e, dh), lambda i: (i, 0)),
            pl.BlockSpec((block_e, 1), lambda i: (i, 0)),
            pl.BlockSpec((dh, hid), full),
            pl.BlockSpec((hid, hid), full),
            pl.BlockSpec((1, hid), full),
            pl.BlockSpec((hid, hid), full),
            pl.BlockSpec((1, hid), full),
            pl.BlockSpec((hid, hid), full),
            pl.BlockSpec((1, hid), full),
            pl.BlockSpec((hid, out_d), full),
            pl.BlockSpec((1, out_d), full),
        ],
        out_specs=pl.BlockSpec((block_e, out_d), lambda i: (i, 0)),
        out_shape=jax.ShapeDtypeStruct((e, out_d), jnp.float32),
    )(pg, hm, dy, W1b, W2, b2.reshape(1, hid), W3, b3.reshape(1, hid), W4,
      b4.reshape(1, hid), W5, b5.reshape(1, out_d))


def _sc_gather(table, idx):
    n, d = table.shape
    e = idx.shape[0]
    epw = e // _NW
    steps = epw // _CH
    pairs = (steps - 1) // 2
    mesh = plsc.VectorSubcoreMesh(
        core_axis_name="c", subcore_axis_name="s", num_cores=_NC,
        num_subcores=_NS)

    @functools.partial(
        pl.kernel,
        out_type=jax.ShapeDtypeStruct((e, d), jnp.float32),
        mesh=mesh,
        scratch_types=[
            pltpu.VMEM((epw,), jnp.int32),
            pltpu.VMEM((_CH, d), jnp.float32),
            pltpu.VMEM((_CH, d), jnp.float32),
            pltpu.SemaphoreType.DMA,
            pltpu.SemaphoreType.DMA,
            pltpu.SemaphoreType.DMA,
        ],
    )
    def gk(table_hbm, idx_hbm, out_hbm, idx_all, rows0, rows1, semg, sems0,
           sems1):
        wid = lax.axis_index("s") * _NC + lax.axis_index("c")
        base = wid * epw
        pltpu.sync_copy(idx_hbm.at[pl.ds(pl.multiple_of(base, _CH), epw)],
                        idx_all)

        def gather(c, buf):
            pltpu.async_copy(table_hbm.at[idx_all.at[pl.ds(c * _CH, _CH)]],
                             buf, semg).wait()

        def store_start(c, buf, sem):
            off = pl.multiple_of(base + c * _CH, _CH)
            return pltpu.async_copy(buf, out_hbm.at[pl.ds(off, _CH)], sem)

        gather(0, rows0)

        def body(i, carry):
            c0 = 2 * i
            s0 = store_start(c0, rows0, sems0)
            gather(c0 + 1, rows1)
            s1 = store_start(c0 + 1, rows1, sems1)
            s0.wait()
            gather(c0 + 2, rows0)
            s1.wait()
            return carry

        lax.fori_loop(0, pairs, body, 0)
        store_start(steps - 1, rows0, sems0).wait()

    return gk(table, idx)


def _sc_scatter(m, idx, n):
    e, d = m.shape
    epw = e // _NW
    steps = epw // _CH
    npt = 640
    n_pad = _NS * npt
    mesh = plsc.VectorSubcoreMesh(
        core_axis_name="c", subcore_axis_name="s", num_cores=_NC,
        num_subcores=_NS)

    pairs = (steps - 1) // 2
    idx3d = idx.reshape(_NW, steps, _CH)

    @functools.partial(
        pl.kernel,
        out_type=(
            jax.ShapeDtypeStruct((_NC, n_pad, d), jnp.float32),
            jax.ShapeDtypeStruct((_NC * n_pad,), jnp.float32),
        ),
        mesh=mesh,
        scratch_types=[
            pltpu.VMEM((steps, _CH), jnp.int32),
            pltpu.VMEM((_CH, d), jnp.float32),
            pltpu.VMEM((_CH, d), jnp.float32),
            pltpu.VMEM((_CH,), jnp.float32),
            pltpu.VMEM((_CH,), jnp.float32),
            pltpu.VMEM_SHARED((n_pad, d), jnp.float32),
            pltpu.VMEM_SHARED((n_pad,), jnp.float32),
            pltpu.SemaphoreType.DMA,
            pltpu.SemaphoreType.DMA,
            pltpu.SemaphoreType.DMA,
            pltpu.SemaphoreType.DMA,
        ],
    )
    def sk(m_hbm, idx_hbm, sum_hbm, cnt_hbm, idx_all, rows0, rows1, ones_v,
           z1_v, table_sh, cnt_sh, semm, sema0, sema1, semo):
        cid = lax.axis_index("c")
        sid = lax.axis_index("s")
        wid = sid * _NC + cid
        base = wid * epw

        zro = jnp.zeros((16,), jnp.float32)
        one = jnp.ones((16,), jnp.float32)

        def fill_body(r, carry):
            for j in range(d // 16):
                rows0[r, pl.ds(j * 16, 16)] = zro
            return carry

        lax.fori_loop(0, _CH, fill_body, 0)
        for j in range(_CH // 16):
            ones_v[pl.ds(j * 16, 16)] = one
            z1_v[pl.ds(j * 16, 16)] = zro

        pltpu.sync_copy(idx_hbm.at[wid], idx_all)

        zbase = sid * npt
        for j in range(npt // _CH):
            pltpu.sync_copy(rows0, table_sh.at[pl.ds(zbase + j * _CH, _CH)])
            pltpu.sync_copy(z1_v, cnt_sh.at[pl.ds(zbase + j * _CH, _CH)])
        plsc.subcore_barrier()

        def load_m(c, buf):
            off = pl.multiple_of(base + c * _CH, 16)
            pltpu.sync_copy(m_hbm.at[pl.ds(off, _CH)], buf)

        def scat_start(c, buf, sem):
            d0 = pltpu.async_copy(buf, table_sh.at[idx_all.at[c]], sem,
                                  add=True)
            d1 = pltpu.async_copy(ones_v, cnt_sh.at[idx_all.at[c]], semo,
                                  add=True)
            return d0, d1

        load_m(0, rows0)

        def body(i, carry):
            c0 = 2 * i
            s0 = scat_start(c0, rows0, sema0)
            load_m(c0 + 1, rows1)
            s1 = scat_start(c0 + 1, rows1, sema1)
            for dsc in s0:
                dsc.wait()
            load_m(c0 + 2, rows0)
            for dsc in s1:
                dsc.wait()
            return carry

        lax.fori_loop(0, pairs, body, 0)
        for dsc in scat_start(steps - 1, rows0, sema0):
            dsc.wait()
        plsc.subcore_barrier()

        for j in range(npt // _CH):
            r0 = zbase + j * _CH
            pltpu.sync_copy(table_sh.at[pl.ds(r0, _CH)], rows0)
            pltpu.sync_copy(rows0, sum_hbm.at[cid, pl.ds(r0, _CH)])
            pltpu.sync_copy(cnt_sh.at[pl.ds(r0, _CH)], z1_v)
            pltpu.sync_copy(z1_v, cnt_hbm.at[pl.ds(cid * n_pad + r0, _CH)])

    return sk(m, idx3d)


def _combine_body(s_ref, c0_ref, c1_ref, o_ref):
    s = s_ref[0] + s_ref[1]
    c = c0_ref[...] + c1_ref[...]
    o_ref[...] = s / jnp.maximum(c, 1.0)


def _combine(sums, cnts, n, block_n=2000):
    d = sums.shape[2]
    n_pad = sums.shape[1]
    c0 = cnts[:n_pad].reshape(n_pad, 1)
    c1 = cnts[n_pad:].reshape(n_pad, 1)
    grid = n // block_n
    return pl.pallas_call(
        _combine_body,
        grid=(grid,),
        in_specs=[
            pl.BlockSpec((2, block_n, d), lambda i: (0, i, 0)),
            pl.BlockSpec((block_n, 1), lambda i: (i, 0)),
            pl.BlockSpec((block_n, 1), lambda i: (i, 0)),
        ],
        out_specs=pl.BlockSpec((block_n, d), lambda i: (i, 0)),
        out_shape=jax.ShapeDtypeStruct((n, d), jnp.float32),
    )(sums, c0, c1)


def kernel(x, edge_index, h_mat_edge, delta_y, Wr, br, W1, b1, W2, b2, W3, b3,
           W4, b4, W5, b5):
    n = x.shape[0]
    d = x.shape[1]
    W1a = W1[:d]
    W1b = W1[d:]

    hid = W1a.shape[1]
    W1a_p = jnp.concatenate([W1a, jnp.zeros((d, d - hid), jnp.float32)], axis=1)
    b1_p = jnp.concatenate([b1, jnp.zeros((d - hid,), jnp.float32)])
    p = _node_proj(x, Wr, br, W1a_p, b1_p)

    dst = edge_index[1]
    pg = _sc_gather(p, dst)

    m = _edge_mlp(pg, h_mat_edge, delta_y, W1b, W2, b2, W3, b3, W4, b4, W5, b5)

    sums, cnts = _sc_scatter(m, dst, n)
    return _combine(sums, cnts, n)


# speedup vs baseline: 1.5262x; 1.0202x over previous
import functools

import jax
import jax.numpy as jnp
from jax import lax
from jax.experimental import pallas as pl
from jax.experimental.pallas import tpu as pltpu
from jax.experimental.pallas import tpu_sc as plsc

_NC = 2
_NS = 16
_NW = _NC * _NS
_CH = 80


def _lrelu(v):
    return jnp.where(v > 0, v, 0.01 * v)


def _node_proj_body(x_ref, wr_ref, br_ref, w1a_ref, b1_ref, o_ref):
    xr = jnp.dot(x_ref[...], wr_ref[...], preferred_element_type=jnp.float32)
    xr = xr + br_ref[...]
    o_ref[...] = (
        jnp.dot(xr, w1a_ref[...], preferred_element_type=jnp.float32) + b1_ref[...]
    )


def _node_proj(x, Wr, br, W1a, b1, block_n=2000):
    n, d = x.shape
    h = W1a.shape[1]
    grid = n // block_n
    full = lambda i: (0, 0)
    return pl.pallas_call(
        _node_proj_body,
        grid=(grid,),
        in_specs=[
            pl.BlockSpec((block_n, d), lambda i: (i, 0)),
            pl.BlockSpec((d, d), full),
            pl.BlockSpec((1, d), full),
            pl.BlockSpec((d, h), full),
            pl.BlockSpec((1, h), full),
        ],
        out_specs=pl.BlockSpec((block_n, h), lambda i: (i, 0)),
        out_shape=jax.ShapeDtypeStruct((n, h), jnp.float32),
    )(x, Wr, br.reshape(1, d), W1a, b1.reshape(1, h))


def _edge_mlp_body(pg_ref, hm_ref, dy_ref, w1b_ref, w2_ref, b2_ref, w3_ref,
                   b3_ref, w4_ref, b4_ref, w5_ref, b5_ref, o_ref):
    f32 = jnp.float32
    hid = w1b_ref.shape[1]
    h1 = _lrelu(
        pg_ref[...][:, :hid]
        + jnp.dot(hm_ref[...], w1b_ref[...], preferred_element_type=f32)
    )
    h2 = _lrelu(jnp.dot(h1, w2_ref[...], preferred_element_type=f32) + b2_ref[...])
    h3 = _lrelu(jnp.dot(h2, w3_ref[...], preferred_element_type=f32) + b3_ref[...])
    h4 = _lrelu(jnp.dot(h3, w4_ref[...], preferred_element_type=f32) + b4_ref[...])
    m = jnp.dot(h4, w5_ref[...], preferred_element_type=f32) + b5_ref[...]
    o_ref[...] = m * dy_ref[...]


def _edge_mlp(pg, hm, dy, W1b, W2, b2, W3, b3, W4, b4, W5, b5, block_e=8000):
    e = hm.shape[0]
    dh = hm.shape[1]
    hid = W2.shape[0]
    out_d = W5.shape[1]
    grid = e // block_e
    full = lambda i: (0, 0)
    return pl.pallas_call(
        _edge_mlp_body,
        grid=(grid,),
        in_specs=[
            pl.BlockSpec((block_e, pg.shape[1]), lambda i: (i, 0)),
            pl.BlockSpec((block_e, dh), lambda i: (i, 0)),
            pl.BlockSpec((block_e, 1), lambda i: (i, 0)),
            pl.BlockSpec((dh, hid), full),
            pl.BlockSpec((hid, hid), full),
            pl.BlockSpec((1, hid), full),
            pl.BlockSpec((hid, hid), full),
            pl.BlockSpec((1, hid), full),
            pl.BlockSpec((hid, hid), full),
            pl.BlockSpec((1, hid), full),
            pl.BlockSpec((hid, out_d), full),
            pl.BlockSpec((1, out_d), full),
        ],
        out_specs=pl.BlockSpec((block_e, out_d), lambda i: (i, 0)),
        out_shape=jax.ShapeDtypeStruct((e, out_d), jnp.float32),
    )(pg, hm, dy, W1b, W2, b2.reshape(1, hid), W3, b3.reshape(1, hid), W4,
      b4.reshape(1, hid), W5, b5.reshape(1, out_d))


def _sc_gather(table, idx):
    n, d = table.shape
    e = idx.shape[0]
    epw = e // _NW
    steps = epw // _CH
    pairs = (steps - 1) // 2
    mesh = plsc.VectorSubcoreMesh(
        core_axis_name="c", subcore_axis_name="s", num_cores=_NC,
        num_subcores=_NS)

    @functools.partial(
        pl.kernel,
        out_type=jax.ShapeDtypeStruct((e, d), jnp.float32),
        mesh=mesh,
        scratch_types=[
            pltpu.VMEM((epw,), jnp.int32),
            pltpu.VMEM((_CH, d), jnp.float32),
            pltpu.VMEM((_CH, d), jnp.float32),
            pltpu.SemaphoreType.DMA,
            pltpu.SemaphoreType.DMA,
            pltpu.SemaphoreType.DMA,
        ],
    )
    def gk(table_hbm, idx_hbm, out_hbm, idx_all, rows0, rows1, semg, sems0,
           sems1):
        wid = lax.axis_index("s") * _NC + lax.axis_index("c")
        base = wid * epw
        pltpu.sync_copy(idx_hbm.at[pl.ds(pl.multiple_of(base, _CH), epw)],
                        idx_all)

        def gather(c, buf):
            pltpu.async_copy(table_hbm.at[idx_all.at[pl.ds(c * _CH, _CH)]],
                             buf, semg).wait()

        def store_start(c, buf, sem):
            off = pl.multiple_of(base + c * _CH, _CH)
            return pltpu.async_copy(buf, out_hbm.at[pl.ds(off, _CH)], sem)

        gather(0, rows0)

        def body(i, carry):
            c0 = 2 * i
            s0 = store_start(c0, rows0, sems0)
            gather(c0 + 1, rows1)
            s1 = store_start(c0 + 1, rows1, sems1)
            s0.wait()
            gather(c0 + 2, rows0)
            s1.wait()
            return carry

        lax.fori_loop(0, pairs, body, 0)
        store_start(steps - 1, rows0, sems0).wait()

    return gk(table, idx)


def _sc_scatter(m, idx, n):
    e, d = m.shape
    epw = e // _NW
    steps = epw // _CH
    npt = 640
    n_pad = _NS * npt
    mesh = plsc.VectorSubcoreMesh(
        core_axis_name="c", subcore_axis_name="s", num_cores=_NC,
        num_subcores=_NS)

    pairs = (steps - 1) // 2
    idx3d = idx.reshape(_NW, steps, _CH)

    @functools.partial(
        pl.kernel,
        out_type=(
            jax.ShapeDtypeStruct((_NC, n_pad, d), jnp.float32),
            jax.ShapeDtypeStruct((_NC * n_pad,), jnp.float32),
        ),
        mesh=mesh,
        scratch_types=[
            pltpu.VMEM((steps, _CH), jnp.int32),
            pltpu.VMEM((_CH, d), jnp.float32),
            pltpu.VMEM((_CH, d), jnp.float32),
            pltpu.VMEM((_CH,), jnp.float32),
            pltpu.VMEM((_CH,), jnp.float32),
            pltpu.VMEM_SHARED((n_pad, d), jnp.float32),
            pltpu.VMEM_SHARED((n_pad,), jnp.float32),
            pltpu.SemaphoreType.DMA,
            pltpu.SemaphoreType.DMA,
            pltpu.SemaphoreType.DMA,
            pltpu.SemaphoreType.DMA,
        ],
    )
    def sk(m_hbm, idx_hbm, sum_hbm, cnt_hbm, idx_all, rows0, rows1, ones_v,
           z1_v, table_sh, cnt_sh, semm, sema0, sema1, semo):
        cid = lax.axis_index("c")
        sid = lax.axis_index("s")
        wid = sid * _NC + cid
        base = wid * epw

        zro = jnp.zeros((16,), jnp.float32)
        one = jnp.ones((16,), jnp.float32)

        def fill_body(r, carry):
            for j in range(d // 16):
                rows0[r, pl.ds(j * 16, 16)] = zro
            return carry

        lax.fori_loop(0, _CH, fill_body, 0)
        for j in range(_CH // 16):
            ones_v[pl.ds(j * 16, 16)] = one
            z1_v[pl.ds(j * 16, 16)] = zro

        pltpu.sync_copy(idx_hbm.at[wid], idx_all)

        zbase = sid * npt
        for j in range(npt // _CH):
            pltpu.sync_copy(rows0, table_sh.at[pl.ds(zbase + j * _CH, _CH)])
            pltpu.sync_copy(z1_v, cnt_sh.at[pl.ds(zbase + j * _CH, _CH)])
        plsc.subcore_barrier()

        def load_m(c, buf):
            off = pl.multiple_of(base + c * _CH, 16)
            pltpu.sync_copy(m_hbm.at[pl.ds(off, _CH)], buf)

        def scat_start(c, buf, sem):
            d0 = pltpu.async_copy(buf, table_sh.at[idx_all.at[c]], sem,
                                  add=True)
            d1 = pltpu.async_copy(ones_v, cnt_sh.at[idx_all.at[c]], semo,
                                  add=True)
            return d0, d1

        load_m(0, rows0)

        def body(i, carry):
            c0 = 2 * i
            s0 = scat_start(c0, rows0, sema0)
            load_m(c0 + 1, rows1)
            s1 = scat_start(c0 + 1, rows1, sema1)
            for dsc in s0:
                dsc.wait()
            load_m(c0 + 2, rows0)
            for dsc in s1:
                dsc.wait()
            return carry

        lax.fori_loop(0, pairs, body, 0)
        for dsc in scat_start(steps - 1, rows0, sema0):
            dsc.wait()
        plsc.subcore_barrier()

        for j in range(npt // _CH):
            r0 = zbase + j * _CH
            pltpu.sync_copy(table_sh.at[pl.ds(r0, _CH)], rows0)
            pltpu.sync_copy(rows0, sum_hbm.at[cid, pl.ds(r0, _CH)])
            pltpu.sync_copy(cnt_sh.at[pl.ds(r0, _CH)], z1_v)
            pltpu.sync_copy(z1_v, cnt_hbm.at[pl.ds(cid * n_pad + r0, _CH)])

    return sk(m, idx3d)


def _combine_body(s_ref, c0_ref, c1_ref, o_ref):
    s = s_ref[0] + s_ref[1]
    c = c0_ref[...] + c1_ref[...]
    o_ref[...] = s / jnp.maximum(c, 1.0)


def _combine(sums, cnts, n, block_n=2000):
    d = sums.shape[2]
    n_pad = sums.shape[1]
    c0 = cnts[:n_pad].reshape(n_pad, 1)
    c1 = cnts[n_pad:].reshape(n_pad, 1)
    grid = n // block_n
    return pl.pallas_call(
        _combine_body,
        grid=(grid,),
        in_specs=[
            pl.BlockSpec((2, block_n, d), lambda i: (0, i, 0)),
            pl.BlockSpec((block_n, 1), lambda i: (i, 0)),
            pl.BlockSpec((block_n, 1), lambda i: (i, 0)),
        ],
        out_specs=pl.BlockSpec((block_n, d), lambda i: (i, 0)),
        out_shape=jax.ShapeDtypeStruct((n, d), jnp.float32),
    )(sums, c0, c1)


def kernel(x, edge_index, h_mat_edge, delta_y, Wr, br, W1, b1, W2, b2, W3, b3,
           W4, b4, W5, b5):
    n = x.shape[0]
    d = x.shape[1]
    W1a = W1[:d]
    W1b = W1[d:]

    hid = W1a.shape[1]
    W1a_p = jnp.concatenate([W1a, jnp.zeros((d, d - hid), jnp.float32)], axis=1)
    b1_p = jnp.concatenate([b1, jnp.zeros((d - hid,), jnp.float32)])
    p = _node_proj(x, Wr, br, W1a_p, b1_p)

    dst = edge_index[1]
    pg = _sc_gather(p, dst)

    m = _edge_mlp(pg, h_mat_edge, delta_y, W1b, W2, b2, W3, b3, W4, b4, W5, b5)

    sums, cnts = _sc_scatter(m, dst, n)
    return _combine(sums, cnts, n)
